# baseline (speedup 1.0000x reference)
import functools

import jax
import jax.numpy as jnp
from jax import lax
from jax.experimental import pallas as pl
from jax.experimental.pallas import tpu as pltpu
from jax.experimental.pallas import tpu_sc as plsc

BATCH = 16384
EMBED_DIM = 32
PACK = 4
PAD_DIM = PACK * EMBED_DIM
NUM_CORES = 2
NUM_SUBCORES = 16
NUM_WORKERS = NUM_CORES * NUM_SUBCORES
ROWS_PER_WORKER = BATCH // NUM_WORKERS
CHUNK = 128
NUM_CHUNKS = ROWS_PER_WORKER // CHUNK
LANES = 16

_sc_mesh = plsc.VectorSubcoreMesh(core_axis_name="c", subcore_axis_name="s")


@functools.partial(
    pl.kernel,
    mesh=_sc_mesh,
    out_type=[
        jax.ShapeDtypeStruct((BATCH, PAD_DIM), jnp.float32),
        jax.ShapeDtypeStruct((BATCH, PAD_DIM), jnp.float32),
    ],
    scratch_types=[
        pltpu.VMEM((ROWS_PER_WORKER,), jnp.int32),
        pltpu.VMEM((ROWS_PER_WORKER,), jnp.int32),
        pltpu.VMEM((2, CHUNK, PAD_DIM), jnp.float32),
        pltpu.VMEM((2, CHUNK, PAD_DIM), jnp.float32),
        pltpu.SemaphoreType.DMA,
        pltpu.SemaphoreType.DMA,
    ],
)
def _sc_gather(uids_hbm, iids_hbm, utab_hbm, itab_hbm, uout_hbm, iout_hbm,
               uidx_v, iidx_v, upad_v, ipad_v, sem_g, sem_o):
    wid = lax.axis_index("s") * NUM_CORES + lax.axis_index("c")
    base = wid * ROWS_PER_WORKER
    pltpu.sync_copy(uids_hbm.at[wid], uidx_v)
    pltpu.sync_copy(iids_hbm.at[wid], iidx_v)
    for g in range(ROWS_PER_WORKER // LANES):
        sl = pl.ds(g * LANES, LANES)
        uidx_v[sl] = lax.shift_right_logical(uidx_v[sl], 2)
        iidx_v[sl] = lax.shift_right_logical(iidx_v[sl], 2)

    def fire(j):
        buf = j % 2
        return (
            pltpu.async_copy(
                utab_hbm.at[uidx_v.at[pl.ds(j * CHUNK, CHUNK)]],
                upad_v.at[buf], sem_g),
            pltpu.async_copy(
                itab_hbm.at[iidx_v.at[pl.ds(j * CHUNK, CHUNK)]],
                ipad_v.at[buf], sem_g),
        )

    def flush(j):
        buf = j % 2
        dst = pl.ds(base + j * CHUNK, CHUNK)
        return (
            pltpu.async_copy(upad_v.at[buf], uout_hbm.at[dst], sem_o),
            pltpu.async_copy(ipad_v.at[buf], iout_hbm.at[dst], sem_o),
        )

    gathers = fire(0)
    outs = []
    for j in range(NUM_CHUNKS):
        for c in gathers:
            c.wait()
        outs.append(flush(j))
        if j + 1 < NUM_CHUNKS:
            if j >= 1:
                for c in outs[j - 1]:
                    c.wait()
            gathers = fire(j + 1)
    for pair in outs[-2:]:
        for c in pair:
            c.wait()


MLP_BLOCK = 2048


def _mlp_body(u_ref, i_ref, uid_ref, iid_ref, w1u_ref, w1i_ref, b1_ref,
              w2_ref, b2_ref, w3_ref, b3_ref, o_ref):
    lane_group = lax.broadcasted_iota(jnp.int32, (MLP_BLOCK, PAD_DIM), 1) // EMBED_DIM
    u_sel = jnp.where(lane_group == (uid_ref[...] & (PACK - 1)), u_ref[...], 0.0)
    i_sel = jnp.where(lane_group == (iid_ref[...] & (PACK - 1)), i_ref[...], 0.0)
    h = jnp.dot(u_sel, w1u_ref[...], preferred_element_type=jnp.float32)
    h = h + jnp.dot(i_sel, w1i_ref[...], preferred_element_type=jnp.float32)
    h = jnp.maximum(h + b1_ref[...], 0.0)
    h = jnp.dot(h, w2_ref[...], preferred_element_type=jnp.float32) + b2_ref[...]
    h = jnp.maximum(h, 0.0)
    o_ref[...] = (
        jnp.dot(h, w3_ref[...], preferred_element_type=jnp.float32) + b3_ref[...]
    )


def _mlp(u_pad, i_pad, uids, iids, W1, b1, W2, b2, W3, b3):
    w1u = jnp.concatenate([W1[:EMBED_DIM]] * PACK, axis=0)
    w1i = jnp.concatenate([W1[EMBED_DIM:]] * PACK, axis=0)
    grid = (BATCH // MLP_BLOCK,)
    full = lambda shape: pl.BlockSpec(shape, lambda i: (0, 0))
    out = pl.pallas_call(
        _mlp_body,
        grid=grid,
        in_specs=[
            pl.BlockSpec((MLP_BLOCK, PAD_DIM), lambda i: (i, 0)),
            pl.BlockSpec((MLP_BLOCK, PAD_DIM), lambda i: (i, 0)),
            pl.BlockSpec((MLP_BLOCK, 1), lambda i: (i, 0)),
            pl.BlockSpec((MLP_BLOCK, 1), lambda i: (i, 0)),
            full(w1u.shape),
            full(w1i.shape),
            full((1, 64)),
            full(W2.shape),
            full((1, 32)),
            full(W3.shape),
            full((1, 1)),
        ],
        out_specs=pl.BlockSpec((MLP_BLOCK, 1), lambda i: (i, 0)),
        out_shape=jax.ShapeDtypeStruct((BATCH, 1), jnp.float32),
    )(u_pad, i_pad, uids.reshape(BATCH, 1), iids.reshape(BATCH, 1), w1u, w1i,
      b1.reshape(1, 64), W2, b2.reshape(1, 32), W3, b3.reshape(1, 1))
    return out[:, 0]


def kernel(user_ids, item_ids, user_table, item_table, W1, b1, W2, b2, W3, b3):
    uids = user_ids.astype(jnp.int32)
    iids = item_ids.astype(jnp.int32)
    utab = user_table.reshape(-1, PAD_DIM)
    itab = item_table.reshape(-1, PAD_DIM)
    u_pad, i_pad = _sc_gather(
        uids.reshape(NUM_WORKERS, ROWS_PER_WORKER),
        iids.reshape(NUM_WORKERS, ROWS_PER_WORKER),
        utab, itab)
    return _mlp(u_pad, i_pad, uids, iids, W1, b1, W2, b2, W3, b3)
